# baseline (speedup 1.0000x reference)
import functools

import jax
import jax.numpy as jnp
from jax import lax
from jax.experimental import pallas as pl
from jax.experimental.pallas import tpu as pltpu
from jax.experimental.pallas import tpu_sc as plsc

B = 16384
D = 16
NC = 2
NS = 16
NW = NC * NS
BPW = B // NW
CH = 128
NCH = BPW // CH
GROUPS = BPW // 16

_mesh = plsc.VectorSubcoreMesh(
    core_axis_name="c", subcore_axis_name="s", num_cores=NC, num_subcores=NS
)


def _nrsqrt(x):
    bits = plsc.bitcast(x, jnp.int32)
    i = jnp.int32(0x5F3759DF) - lax.shift_right_logical(bits, 1)
    y = plsc.bitcast(i, jnp.float32)
    for _ in range(3):
        y = y * (1.5 - 0.5 * x * y * y)
    return y


@functools.partial(
    pl.kernel,
    out_type=jax.ShapeDtypeStruct((B,), jnp.float32),
    mesh=_mesh,
    compiler_params=pltpu.CompilerParams(
        needs_layout_passes=False, use_tc_tiling_on_sc=False),
    scratch_types=[
        pltpu.VMEM((NCH, CH), jnp.int32),
        pltpu.VMEM((NCH, CH), jnp.int32),
        pltpu.VMEM((BPW, D), jnp.float32),
        pltpu.VMEM((BPW, D), jnp.float32),
        pltpu.VMEM((BPW,), jnp.float32),
        pltpu.SemaphoreType.DMA,
    ],
)
def _sc_cosine(uidx_hbm, midx_hbm, utab_hbm, mtab_hbm, out_hbm,
               uidx_v, midx_v, urows_v, mrows_v, out_v, sem):
    wid = lax.axis_index("s") * NC + lax.axis_index("c")

    pltpu.sync_copy(uidx_hbm.at[wid], uidx_v)
    pltpu.sync_copy(midx_hbm.at[wid], midx_v)

    copies = []
    for ch in range(NCH):
        copies.append(pltpu.async_copy(
            utab_hbm.at[uidx_v.at[ch]], urows_v.at[pl.ds(ch * CH, CH)], sem))
        copies.append(pltpu.async_copy(
            mtab_hbm.at[midx_v.at[ch]], mrows_v.at[pl.ds(ch * CH, CH)], sem))
    for cp in copies:
        cp.wait()

    def group(g, carry):
        rbase = g * 16
        rows = rbase + lax.iota(jnp.int32, 16)
        uu = jnp.zeros((16,), jnp.float32)
        mm = jnp.zeros((16,), jnp.float32)
        um = jnp.zeros((16,), jnp.float32)
        for j in range(D):
            col = jnp.full((16,), j, jnp.int32)
            uc = plsc.load_gather(urows_v, [rows, col])
            mc = plsc.load_gather(mrows_v, [rows, col])
            uu = uu + uc * uc
            mm = mm + mc * mc
            um = um + uc * mc
        un = jnp.maximum(uu * _nrsqrt(uu), 1e-8)
        mn = jnp.maximum(mm * _nrsqrt(mm), 1e-8)
        sim = um / (un * mn) * 2.5 + 2.75
        out_v[pl.ds(rbase, 16)] = sim
        return carry

    lax.fori_loop(0, GROUPS, group, 0)
    pltpu.sync_copy(out_v, out_hbm.at[pl.ds(wid * BPW, BPW)])


def kernel(user_idx, movie_idx, user_table, movie_table):
    uidx = user_idx.astype(jnp.int32).reshape(NW, NCH, CH)
    midx = movie_idx.astype(jnp.int32).reshape(NW, NCH, CH)
    return _sc_cosine(uidx, midx, user_table, movie_table)


# speedup vs baseline: 7.1424x; 7.1424x over previous
import functools

import jax
import jax.numpy as jnp
from jax import lax
from jax.experimental import pallas as pl
from jax.experimental.pallas import tpu as pltpu
from jax.experimental.pallas import tpu_sc as plsc

B = 16384
D = 16
NC = 2
NS = 16
NW = NC * NS
BPW = B // NW
GROUPS = BPW // 16

_mesh = plsc.VectorSubcoreMesh(
    core_axis_name="c", subcore_axis_name="s", num_cores=NC, num_subcores=NS
)

_IOTA = lambda: lax.iota(jnp.int32, 16)


def _lane(vec, t):
    return lax.reduce_sum(
        jnp.where(_IOTA() == t, vec, jnp.int32(0)), axes=(0,))


def _nrsqrt(x):
    bits = plsc.bitcast(x, jnp.int32)
    i = jnp.int32(0x5F3759DF) - lax.shift_right_logical(bits, 1)
    y = plsc.bitcast(i, jnp.float32)
    for _ in range(3):
        y = y * (1.5 - 0.5 * x * y * y)
    return y


@functools.partial(
    pl.kernel,
    out_type=jax.ShapeDtypeStruct((B,), jnp.float32),
    mesh=_mesh,
    compiler_params=pltpu.CompilerParams(
        needs_layout_passes=False, disable_bounds_checks=True),
    scratch_types=[
        pltpu.VMEM((BPW,), jnp.int32),
        pltpu.VMEM((BPW,), jnp.int32),
        pltpu.VMEM((16, D, 128), jnp.float32),
        pltpu.VMEM((16, D, 128), jnp.float32),
        pltpu.VMEM((D, BPW), jnp.float32),
        pltpu.VMEM((D, BPW), jnp.float32),
        pltpu.VMEM((BPW,), jnp.float32),
        pltpu.SemaphoreType.DMA,
        pltpu.SemaphoreType.DMA,
    ],
)
def _sc_cosine(uidx_hbm, midx_hbm, utab_hbm, mtab_hbm, out_hbm,
               uidx_v, midx_v, uring_v, mring_v, ucols_v, mcols_v,
               out_v, sem_u, sem_m):
    wid = lax.axis_index("s") * NC + lax.axis_index("c")

    pltpu.sync_copy(uidx_hbm.at[wid], uidx_v)
    pltpu.sync_copy(midx_hbm.at[wid], midx_v)

    def slab_bases(base):
        uq = jnp.bitwise_and(uidx_v[pl.ds(base, 16)], -128)
        mq = jnp.bitwise_and(midx_v[pl.ds(base, 16)], -128)
        uqs = [pl.multiple_of(_lane(uq, t), 128) for t in range(16)]
        mqs = [pl.multiple_of(_lane(mq, t), 128) for t in range(16)]
        return uqs, mqs

    def fire(t, qu, qm):
        pltpu.async_copy(utab_hbm.at[:, pl.ds(qu, 128)], uring_v.at[t], sem_u)
        pltpu.async_copy(mtab_hbm.at[:, pl.ds(qm, 128)], mring_v.at[t], sem_m)

    uqs0, mqs0 = slab_bases(0)
    for t in range(16):
        fire(t, uqs0[t], mqs0[t])

    def block(g, carry):
        nb = g + 1
        nbase = jnp.minimum(nb, GROUPS - 1) * 16
        uqs, mqs = slab_bases(nbase)
        base = g * 16
        ul = jnp.bitwise_and(uidx_v[pl.ds(base, 16)], 127)
        ml = jnp.bitwise_and(midx_v[pl.ds(base, 16)], 127)
        for t in range(16):
            pltpu.make_async_copy(
                utab_hbm.at[:, pl.ds(0, 128)], uring_v.at[t], sem_u).wait()
            pltpu.make_async_copy(
                mtab_hbm.at[:, pl.ds(0, 128)], mring_v.at[t], sem_m).wait()
            usp = jnp.full((16,), _lane(ul, t), jnp.int32)
            msp = jnp.full((16,), _lane(ml, t), jnp.int32)
            urow = plsc.load_gather(uring_v.at[t], [_IOTA(), usp])
            mrow = plsc.load_gather(mring_v.at[t], [_IOTA(), msp])
            kv = jnp.full((16,), base + t, jnp.int32)
            plsc.store_scatter(ucols_v, [_IOTA(), kv], urow)
            plsc.store_scatter(mcols_v, [_IOTA(), kv], mrow)

            @pl.when(nb < GROUPS)
            def _():
                fire(t, uqs[t], mqs[t])

        rows = base + _IOTA()
        uu = jnp.zeros((16,), jnp.float32)
        mm = jnp.zeros((16,), jnp.float32)
        um = jnp.zeros((16,), jnp.float32)
        for j in range(D):
            jv = jnp.full((16,), j, jnp.int32)
            uc = plsc.load_gather(ucols_v, [jv, rows])
            mc = plsc.load_gather(mcols_v, [jv, rows])
            uu = uu + uc * uc
            mm = mm + mc * mc
            um = um + uc * mc
        un = jnp.maximum(uu * _nrsqrt(uu), 1e-8)
        mn = jnp.maximum(mm * _nrsqrt(mm), 1e-8)
        sim = um / (un * mn) * 2.5 + 2.75
        out_v[pl.ds(base, 16)] = sim
        return carry

    lax.fori_loop(0, GROUPS, block, 0)
    pltpu.sync_copy(out_v, out_hbm.at[pl.ds(wid * BPW, BPW)])


def kernel(user_idx, movie_idx, user_table, movie_table):
    uidx = user_idx.astype(jnp.int32).reshape(NW, BPW)
    midx = movie_idx.astype(jnp.int32).reshape(NW, BPW)
    ut = jnp.swapaxes(user_table, 0, 1)
    mt = jnp.swapaxes(movie_table, 0, 1)
    return _sc_cosine(uidx, midx, ut, mt)


# speedup vs baseline: 7.1479x; 1.0008x over previous
import functools

import jax
import jax.numpy as jnp
from jax import lax
from jax.experimental import pallas as pl
from jax.experimental.pallas import tpu as pltpu
from jax.experimental.pallas import tpu_sc as plsc

B = 16384
D = 16
NC = 2
NS = 16
NW = NC * NS
BPW = B // NW
GROUPS = BPW // 16

_mesh = plsc.VectorSubcoreMesh(
    core_axis_name="c", subcore_axis_name="s", num_cores=NC, num_subcores=NS
)

_IOTA = lambda: lax.iota(jnp.int32, 16)


def _lane(vec, t):
    return lax.reduce_sum(
        jnp.where(_IOTA() == t, vec, jnp.int32(0)), axes=(0,))


def _nrsqrt(x):
    bits = plsc.bitcast(x, jnp.int32)
    i = jnp.int32(0x5F3759DF) - lax.shift_right_logical(bits, 1)
    y = plsc.bitcast(i, jnp.float32)
    for _ in range(3):
        y = y * (1.5 - 0.5 * x * y * y)
    return y


@functools.partial(
    pl.kernel,
    out_type=jax.ShapeDtypeStruct((B,), jnp.float32),
    mesh=_mesh,
    compiler_params=pltpu.CompilerParams(
        needs_layout_passes=False, disable_bounds_checks=True),
    scratch_types=[
        pltpu.VMEM((BPW,), jnp.int32),
        pltpu.VMEM((BPW,), jnp.int32),
        pltpu.VMEM((16, D, 128), jnp.float32),
        pltpu.VMEM((16, D, 128), jnp.float32),
        pltpu.VMEM((D, BPW), jnp.float32),
        pltpu.VMEM((D, BPW), jnp.float32),
        pltpu.VMEM((BPW,), jnp.float32),
        pltpu.VMEM((2, 16), jnp.int32),
        pltpu.SemaphoreType.DMA,
        pltpu.SemaphoreType.DMA,
    ],
)
def _sc_cosine(uidx_hbm, midx_hbm, utab_hbm, mtab_hbm, out_hbm,
               uidx_v, midx_v, uring_v, mring_v, ucols_v, mcols_v,
               out_v, lbuf_v, sem_u, sem_m):
    wid = lax.axis_index("s") * NC + lax.axis_index("c")

    pltpu.sync_copy(uidx_hbm.at[wid], uidx_v)
    pltpu.sync_copy(midx_hbm.at[wid], midx_v)

    def slab_bases(base):
        uq = jnp.bitwise_and(uidx_v[pl.ds(base, 16)], -128)
        mq = jnp.bitwise_and(midx_v[pl.ds(base, 16)], -128)
        uqs = [pl.multiple_of(_lane(uq, t), 128) for t in range(16)]
        mqs = [pl.multiple_of(_lane(mq, t), 128) for t in range(16)]
        return uqs, mqs

    def fire(t, qu, qm):
        pltpu.async_copy(utab_hbm.at[:, pl.ds(qu, 128)], uring_v.at[t], sem_u)
        pltpu.async_copy(mtab_hbm.at[:, pl.ds(qm, 128)], mring_v.at[t], sem_m)

    uqs0, mqs0 = slab_bases(0)
    for t in range(16):
        fire(t, uqs0[t], mqs0[t])

    def block(g, carry):
        nb = g + 1
        nbase = jnp.minimum(nb, GROUPS - 1) * 16
        uqs, mqs = slab_bases(nbase)
        base = g * 16
        lbuf_v[0, :] = jnp.bitwise_and(uidx_v[pl.ds(base, 16)], 127)
        lbuf_v[1, :] = jnp.bitwise_and(midx_v[pl.ds(base, 16)], 127)
        for t in range(16):
            pltpu.make_async_copy(
                utab_hbm.at[:, pl.ds(0, 128)], uring_v.at[t], sem_u).wait()
            pltpu.make_async_copy(
                mtab_hbm.at[:, pl.ds(0, 128)], mring_v.at[t], sem_m).wait()
            tsplat = jnp.full((16,), t, jnp.int32)
            usp = plsc.load_gather(lbuf_v, [jnp.zeros((16,), jnp.int32), tsplat])
            msp = plsc.load_gather(lbuf_v, [jnp.ones((16,), jnp.int32), tsplat])
            urow = plsc.load_gather(uring_v.at[t], [_IOTA(), usp])
            mrow = plsc.load_gather(mring_v.at[t], [_IOTA(), msp])
            kv = jnp.full((16,), base + t, jnp.int32)
            plsc.store_scatter(ucols_v, [_IOTA(), kv], urow)
            plsc.store_scatter(mcols_v, [_IOTA(), kv], mrow)

            @pl.when(nb < GROUPS)
            def _():
                fire(t, uqs[t], mqs[t])

        rows = base + _IOTA()
        uu = jnp.zeros((16,), jnp.float32)
        mm = jnp.zeros((16,), jnp.float32)
        um = jnp.zeros((16,), jnp.float32)
        for j in range(D):
            jv = jnp.full((16,), j, jnp.int32)
            uc = plsc.load_gather(ucols_v, [jv, rows])
            mc = plsc.load_gather(mcols_v, [jv, rows])
            uu = uu + uc * uc
            mm = mm + mc * mc
            um = um + uc * mc
        un = jnp.maximum(uu * _nrsqrt(uu), 1e-8)
        mn = jnp.maximum(mm * _nrsqrt(mm), 1e-8)
        sim = um / (un * mn) * 2.5 + 2.75
        out_v[pl.ds(base, 16)] = sim
        return carry

    lax.fori_loop(0, GROUPS, block, 0)
    pltpu.sync_copy(out_v, out_hbm.at[pl.ds(wid * BPW, BPW)])


def kernel(user_idx, movie_idx, user_table, movie_table):
    uidx = user_idx.astype(jnp.int32).reshape(NW, BPW)
    midx = movie_idx.astype(jnp.int32).reshape(NW, BPW)
    ut = jnp.swapaxes(user_table, 0, 1)
    mt = jnp.swapaxes(movie_table, 0, 1)
    return _sc_cosine(uidx, midx, ut, mt)
